# baseline (speedup 1.0000x reference)
import functools

import jax
import jax.numpy as jnp
from jax import lax
from jax.experimental import pallas as pl
from jax.experimental.pallas import tpu as pltpu
from jax.experimental.pallas import tpu_sc as plsc

_NUM_CORES = 2
_NUM_SUBCORES = 16
_NUM_WORKERS = _NUM_CORES * _NUM_SUBCORES
_LANES = 16

_GATHER_SLICE = 128


def _make_sc_kernel(n_nodes, n_subtok, emb_dim, nodes_per_chunk):
  assert n_nodes % _NUM_WORKERS == 0
  nodes_per_worker = n_nodes // _NUM_WORKERS
  assert nodes_per_worker % nodes_per_chunk == 0
  num_chunks = nodes_per_worker // nodes_per_chunk
  assert num_chunks % 2 == 0
  idx_per_chunk = nodes_per_chunk * n_subtok
  assert idx_per_chunk % _GATHER_SLICE == 0
  gathers_per_chunk = idx_per_chunk // _GATHER_SLICE
  assert emb_dim % _LANES == 0
  groups = emb_dim // _LANES
  scale = 1.0 / float(n_nodes)

  mesh = plsc.VectorSubcoreMesh(
      core_axis_name="c", subcore_axis_name="s",
      num_cores=_NUM_CORES, num_subcores=_NUM_SUBCORES)

  @functools.partial(
      pl.kernel,
      out_type=jax.ShapeDtypeStruct((n_nodes, emb_dim), jnp.float32),
      mesh=mesh,
      scratch_types=[
          pltpu.VMEM((nodes_per_chunk, n_subtok), jnp.int32),
          pltpu.VMEM((nodes_per_chunk, n_subtok), jnp.int32),
          pltpu.VMEM((idx_per_chunk, emb_dim), jnp.float32),
          pltpu.VMEM((idx_per_chunk, emb_dim), jnp.float32),
          pltpu.VMEM((nodes_per_chunk, emb_dim), jnp.float32),
          pltpu.SemaphoreType.DMA,
          pltpu.SemaphoreType.DMA,
      ],
      compiler_params=pltpu.CompilerParams(use_tc_tiling_on_sc=False),
  )
  def sc_kernel(x_hbm, table_hbm, out_hbm,
                idx0, idx1, rows0, rows1, out_v, sem0, sem1):
    wid = lax.axis_index("s") * _NUM_CORES + lax.axis_index("c")
    node_base = wid * nodes_per_worker

    def fire(chunk, idx_v, rows_v, sem):
      base = node_base + chunk * nodes_per_chunk
      pltpu.sync_copy(x_hbm.at[pl.ds(base, nodes_per_chunk), :], idx_v)
      for i in range(nodes_per_chunk):
        pltpu.async_copy(table_hbm.at[idx_v.at[i, :]],
                         rows_v.at[pl.ds(i * n_subtok, n_subtok)], sem)

    def drain(idx_v, rows_v, sem):
      for i in range(nodes_per_chunk):
        pltpu.make_async_copy(
            table_hbm.at[idx_v.at[i, :]],
            rows_v.at[pl.ds(i * n_subtok, n_subtok)], sem).wait()

    def compute_store(chunk, rows_v):
      def node_body(n, _):
        row = n * n_subtok
        for g in range(groups):
          dsl = pl.ds(g * _LANES, _LANES)
          acc = rows_v[row, dsl]
          for j in range(1, n_subtok):
            acc = acc + rows_v[row + j, dsl]
          out_v[n, dsl] = acc * scale
        return 0

      lax.fori_loop(0, nodes_per_chunk, node_body, 0)
      out_base = node_base + chunk * nodes_per_chunk
      pltpu.sync_copy(out_v, out_hbm.at[pl.ds(out_base, nodes_per_chunk)])

    fire(0, idx0, rows0, sem0)

    def loop_body(i, _):
      c = i * 2
      fire(c + 1, idx1, rows1, sem1)
      drain(idx0, rows0, sem0)
      compute_store(c, rows0)
      fire(c + 2, idx0, rows0, sem0)
      drain(idx1, rows1, sem1)
      compute_store(c + 1, rows1)
      return 0

    lax.fori_loop(0, num_chunks // 2 - 1, loop_body, 0)

    last = num_chunks - 2
    fire(last + 1, idx1, rows1, sem1)
    drain(idx0, rows0, sem0)
    compute_store(last, rows0)
    drain(idx1, rows1, sem1)
    compute_store(last + 1, rows1)

  return sc_kernel


def kernel(x, emb_table):
  n_nodes, n_subtok = x.shape
  _, emb_dim = emb_table.shape
  sc = _make_sc_kernel(n_nodes, n_subtok, emb_dim, nodes_per_chunk=32)
  return sc(x.astype(jnp.int32), emb_table.astype(jnp.float32))


# speedup vs baseline: 1.3183x; 1.3183x over previous
import functools

import jax
import jax.numpy as jnp
from jax import lax
from jax.experimental import pallas as pl
from jax.experimental.pallas import tpu as pltpu
from jax.experimental.pallas import tpu_sc as plsc

_NUM_CORES = 2
_NUM_SUBCORES = 16
_NUM_WORKERS = _NUM_CORES * _NUM_SUBCORES
_LANES = 16

_GATHER_SLICE = 128


def _make_sc_kernel(n_nodes, n_subtok, emb_dim, nodes_per_chunk):
  assert n_nodes % _NUM_WORKERS == 0
  nodes_per_worker = n_nodes // _NUM_WORKERS
  assert nodes_per_worker % nodes_per_chunk == 0
  num_chunks = nodes_per_worker // nodes_per_chunk
  assert num_chunks % 2 == 0
  idx_per_chunk = nodes_per_chunk * n_subtok
  assert idx_per_chunk % _GATHER_SLICE == 0
  gathers_per_chunk = idx_per_chunk // _GATHER_SLICE
  assert emb_dim % _LANES == 0
  groups = emb_dim // _LANES
  scale = 1.0 / float(n_nodes)

  mesh = plsc.VectorSubcoreMesh(
      core_axis_name="c", subcore_axis_name="s",
      num_cores=_NUM_CORES, num_subcores=_NUM_SUBCORES)

  @functools.partial(
      pl.kernel,
      out_type=jax.ShapeDtypeStruct((n_nodes, emb_dim), jnp.float32),
      mesh=mesh,
      scratch_types=[
          pltpu.VMEM((n_subtok, nodes_per_chunk), jnp.int32),
          pltpu.VMEM((n_subtok, nodes_per_chunk), jnp.int32),
          pltpu.VMEM((idx_per_chunk, emb_dim), jnp.float32),
          pltpu.VMEM((idx_per_chunk, emb_dim), jnp.float32),
          pltpu.VMEM((nodes_per_chunk, emb_dim), jnp.float32),
          pltpu.SemaphoreType.DMA,
          pltpu.SemaphoreType.DMA,
      ],
      compiler_params=pltpu.CompilerParams(use_tc_tiling_on_sc=False),
  )
  def sc_kernel(x_hbm, table_hbm, out_hbm,
                idx0, idx1, rows0, rows1, out_v, sem0, sem1):
    wid = lax.axis_index("s") * _NUM_CORES + lax.axis_index("c")
    node_base = wid * nodes_per_worker

    def fire(chunk, idx_v, rows_v, sem):
      base = node_base + chunk * nodes_per_chunk
      pltpu.sync_copy(x_hbm.at[:, pl.ds(base, nodes_per_chunk)], idx_v)
      for j in range(n_subtok):
        pltpu.async_copy(table_hbm.at[idx_v.at[j, :]],
                         rows_v.at[pl.ds(j * nodes_per_chunk, nodes_per_chunk)],
                         sem)

    def drain(idx_v, rows_v, sem):
      for j in range(n_subtok):
        pltpu.make_async_copy(
            table_hbm.at[idx_v.at[j, :]],
            rows_v.at[pl.ds(j * nodes_per_chunk, nodes_per_chunk)], sem).wait()

    def compute_store(chunk, rows_v):
      def node_body(n, _):
        for g in range(groups):
          dsl = pl.ds(g * _LANES, _LANES)
          acc = rows_v[n, dsl]
          for j in range(1, n_subtok):
            acc = acc + rows_v[j * nodes_per_chunk + n, dsl]
          out_v[n, dsl] = acc * scale
        return 0

      lax.fori_loop(0, nodes_per_chunk, node_body, 0)
      out_base = node_base + chunk * nodes_per_chunk
      pltpu.sync_copy(out_v, out_hbm.at[pl.ds(out_base, nodes_per_chunk)])

    fire(0, idx0, rows0, sem0)

    def loop_body(i, _):
      c = i * 2
      fire(c + 1, idx1, rows1, sem1)
      drain(idx0, rows0, sem0)
      compute_store(c, rows0)
      fire(c + 2, idx0, rows0, sem0)
      drain(idx1, rows1, sem1)
      compute_store(c + 1, rows1)
      return 0

    lax.fori_loop(0, num_chunks // 2 - 1, loop_body, 0)

    last = num_chunks - 2
    fire(last + 1, idx1, rows1, sem1)
    drain(idx0, rows0, sem0)
    compute_store(last, rows0)
    drain(idx1, rows1, sem1)
    compute_store(last + 1, rows1)

  return sc_kernel


def kernel(x, emb_table):
  n_nodes, n_subtok = x.shape
  _, emb_dim = emb_table.shape
  sc = _make_sc_kernel(n_nodes, n_subtok, emb_dim, nodes_per_chunk=32)
  return sc(x.T.astype(jnp.int32), emb_table.astype(jnp.float32))
